# Initial kernel scaffold; baseline (speedup 1.0000x reference)
import jax
import jax.numpy as jnp
from jax.experimental import pallas as pl


def kernel(x, W_lin, bias, ffm_tables):
    raise NotImplementedError("write your pallas kernel here")


# speedup vs baseline: 11.4108x; 11.4108x over previous
import functools

import jax
import jax.numpy as jnp
import numpy as np
from jax import lax
from jax.experimental import pallas as pl
from jax.experimental.pallas import tpu as pltpu
from jax.experimental.pallas import tpu_sc as plsc

F = 26
D = 32
B = 1024
ROWS = F * F
ROWS_PAD = 680
L = 16

NC, NS = 2, 16
NW = NC * NS
B_PER_W = B // NW

CHUNKS = [(c * 128, min(128, ROWS_PAD - c * 128)) for c in range((ROWS_PAD + 127) // 128)]

_II, _JJ = np.triu_indices(F, k=1)
PAIRS = [(int(i), int(j)) for i, j in zip(_II, _JJ)]


def _ffm_body(idx_hbm, wl_hbm, bias_hbm, table_hbm, out_hbm,
              wl_v, idxrow_v, rows_v, out_v, bias_v, sem):
    wid = lax.axis_index("s") * NC + lax.axis_index("c")
    base_b = wid * B_PER_W

    pltpu.sync_copy(wl_hbm, wl_v)
    pltpu.sync_copy(bias_hbm, bias_v)
    bias_vec = bias_v[...]

    lane = lax.iota(jnp.int32, L)
    lin_mask = lane < (F - L)
    lin_maskf = lin_mask.astype(jnp.float32)

    def body(bb, lanevec):
        b = base_b + bb
        pltpu.sync_copy(idx_hbm.at[b], idxrow_v)
        copies = []
        for off, n in CHUNKS:
            copies.append(pltpu.async_copy(
                table_hbm.at[idxrow_v.at[pl.ds(off, n)]],
                rows_v.at[pl.ds(off, n)], sem))
        for c in copies:
            c.wait()

        acc0 = jnp.zeros((L,), jnp.float32)
        acc1 = jnp.zeros((L,), jnp.float32)
        for i, j in PAIRS:
            a = i * F + j
            p = j * F + i
            acc0 = acc0 + rows_v[a, pl.ds(0, L)] * rows_v[p, pl.ds(0, L)]
            acc1 = acc1 + rows_v[a, pl.ds(L, L)] * rows_v[p, pl.ds(L, L)]

        ridx0 = idxrow_v[pl.ds(0, L)]
        ridx1 = jnp.where(lin_mask, idxrow_v[pl.ds(L, L)], 0)
        lin0 = plsc.load_gather(wl_v, [ridx0])
        lin1 = plsc.load_gather(wl_v, [ridx1]) * lin_maskf

        total = jnp.sum(acc0 + acc1 + lin0 + lin1)

        lanevec = jnp.where(lane == (bb % L), total, lanevec)

        @pl.when(bb % L == L - 1)
        def _():
            s = lanevec + bias_vec
            sig = 1.0 / (1.0 + jnp.exp(-s))
            out_v[bb // L] = sig

        return lanevec

    lax.fori_loop(0, B_PER_W, body, jnp.zeros((L,), jnp.float32), unroll=False)

    pltpu.sync_copy(out_v, out_hbm.at[pl.ds(wid * (B_PER_W // L), B_PER_W // L)])


@jax.jit
def _ffm_sc(idx_pad, wl_flat, bias_bcast, table_flat):
    kfn = functools.partial(
        pl.kernel,
        out_type=jax.ShapeDtypeStruct((B // L, L), jnp.float32),
        mesh=plsc.VectorSubcoreMesh(core_axis_name="c", subcore_axis_name="s"),
        compiler_params=pltpu.CompilerParams(use_tc_tiling_on_sc=False, needs_layout_passes=False),
        scratch_types=[
            pltpu.VMEM((26000,), jnp.float32),
            pltpu.VMEM((ROWS_PAD,), jnp.int32),
            pltpu.VMEM((ROWS_PAD, D), jnp.float32),
            pltpu.VMEM((B_PER_W // L, L), jnp.float32),
            pltpu.VMEM((L,), jnp.float32),
            pltpu.SemaphoreType.DMA,
        ],
    )(_ffm_body)
    return kfn(idx_pad, wl_flat, bias_bcast, table_flat)


def kernel(x, W_lin, bias, ffm_tables):
    f = x.shape[1]
    total = W_lin.shape[0]
    offsets = (jnp.arange(f, dtype=x.dtype) * (total // f))[None, :]
    idx = x + offsets
    g = (idx[:, None, :] + (jnp.arange(f, dtype=x.dtype) * total)[None, :, None])
    g = g.reshape(x.shape[0], f * f)
    g = jnp.pad(g, ((0, 0), (0, ROWS_PAD - f * f)))
    out = _ffm_sc(g, W_lin.reshape(-1),
                  jnp.broadcast_to(bias, (L,)),
                  ffm_tables.reshape(-1, ffm_tables.shape[-1]))
    return out.reshape(-1)
